# baseline (speedup 1.0000x reference)
import functools

import jax
import jax.numpy as jnp
from jax import lax
from jax.experimental import pallas as pl
from jax.experimental.pallas import tpu as pltpu
from jax.experimental.pallas import tpu_sc as plsc

_LANES = 16


def _make_pool(batch, ctx, voc, emb):
    info = plsc.get_sparse_core_info()
    nw = info.num_cores * info.num_subcores
    b_per_w = batch // nw
    idx_per_w = b_per_w * ctx
    nc = info.num_cores
    scale = jnp.float32(1.0 / ctx)

    def body(idx_hbm, table_hbm, out_hbm, idx_v, rows_v, pooled_v, sem):
        wid = lax.axis_index("s") * nc + lax.axis_index("c")
        ibase = wid * idx_per_w
        bbase = wid * b_per_w
        pltpu.sync_copy(idx_hbm.at[pl.ds(ibase, idx_per_w)], idx_v)
        pltpu.async_copy(table_hbm.at[idx_v], rows_v, sem).wait()

        def row_body(b, carry):
            def ctx_body(c, accs):
                a0, a1 = accs
                r = b * ctx + c
                return (a0 + rows_v[r, 0:_LANES],
                        a1 + rows_v[r, _LANES:2 * _LANES])

            a0, a1 = lax.fori_loop(
                0, ctx, ctx_body,
                (jnp.zeros((_LANES,), jnp.float32),
                 jnp.zeros((_LANES,), jnp.float32)))
            pooled_v[b, 0:_LANES] = a0 * scale
            pooled_v[b, _LANES:2 * _LANES] = a1 * scale
            return carry

        lax.fori_loop(0, b_per_w, row_body, 0)
        pltpu.sync_copy(pooled_v, out_hbm.at[pl.ds(bbase, b_per_w)])

    return pl.kernel(
        body,
        mesh=plsc.VectorSubcoreMesh(core_axis_name="c", subcore_axis_name="s"),
        compiler_params=pltpu.CompilerParams(use_tc_tiling_on_sc=False),
        out_type=jax.ShapeDtypeStruct((batch, emb), jnp.float32),
        scratch_types=[
            pltpu.VMEM((idx_per_w,), jnp.int32),
            pltpu.VMEM((idx_per_w, emb), jnp.float32),
            pltpu.VMEM((b_per_w, emb), jnp.float32),
            pltpu.SemaphoreType.DMA,
        ],
    )


_NBUF = 4


def _make_proj_body(vblk, nstep, tail):
    def body(p_ref, w_ref, b_ref, o_hbm, obuf, sems):
        i = pl.program_id(0)
        slot = lax.rem(i, _NBUF)

        @pl.when(i >= _NBUF)
        def _wait_prev():
            pltpu.make_async_copy(
                obuf.at[slot],
                o_hbm.at[:, pl.ds((i - _NBUF) * vblk, vblk)],
                sems.at[slot]).wait()

        obuf[slot] = lax.dot_general(
            p_ref[...], w_ref[...], (((1,), (1,)), ((), ())),
            preferred_element_type=jnp.float32) + b_ref[...]

        @pl.when(i < nstep - 1)
        def _fire():
            pltpu.make_async_copy(
                obuf.at[slot],
                o_hbm.at[:, pl.ds(i * vblk, vblk)],
                sems.at[slot]).start()

        @pl.when(i == nstep - 1)
        def _fire_tail_and_drain():
            pltpu.make_async_copy(
                obuf.at[slot, :, 0:tail],
                o_hbm.at[:, pl.ds((nstep - 1) * vblk, tail)],
                sems.at[slot]).start()
            for d in range(1, _NBUF):
                j = i - _NBUF + d
                sj = lax.rem(j, _NBUF)
                pltpu.make_async_copy(
                    obuf.at[sj],
                    o_hbm.at[:, pl.ds(j * vblk, vblk)],
                    sems.at[sj]).wait()
            pltpu.make_async_copy(
                obuf.at[slot, :, 0:tail],
                o_hbm.at[:, pl.ds((nstep - 1) * vblk, tail)],
                sems.at[slot]).wait()

    return body


def _dot_bias(p_ref, w_ref, b_ref):
    return lax.dot_general(
        p_ref[...], w_ref[...], (((1,), (1,)), ((), ())),
        preferred_element_type=jnp.float32) + b_ref[...]


def _make_main_body(vblk, nstep, nbuf):
    def body(p_ref, w_ref, b_ref, o_hbm, obuf, sems):
        i = pl.program_id(0)
        slot = lax.rem(i, nbuf)

        @pl.when(i >= nbuf)
        def _wait_prev():
            pltpu.make_async_copy(
                obuf.at[slot],
                o_hbm.at[:, pl.ds((i - nbuf) * vblk, vblk)],
                sems.at[slot]).wait()

        obuf[slot] = _dot_bias(p_ref, w_ref, b_ref)
        pltpu.make_async_copy(
            obuf.at[slot],
            o_hbm.at[:, pl.ds(i * vblk, vblk)],
            sems.at[slot]).start()

        @pl.when(i == nstep - 1)
        def _drain():
            for d in range(nbuf - 1, -1, -1):
                j = i - d
                sj = lax.rem(j, nbuf)
                pltpu.make_async_copy(
                    obuf.at[sj],
                    o_hbm.at[:, pl.ds(j * vblk, vblk)],
                    sems.at[sj]).wait()

    return body


def _tail_body(o_in_ref, p_ref, w_ref, b_ref, o_ref):
    del o_in_ref
    o_ref[...] = _dot_bias(p_ref, w_ref, b_ref)


def _project(pooled, lin_w, lin_b, vblk, nbuf):
    batch, emb = pooled.shape
    voc = lin_w.shape[0]
    nstep = voc // vblk
    bias2d = lin_b.reshape(1, voc)
    main = pl.pallas_call(
        _make_main_body(vblk, nstep, nbuf),
        grid=(nstep,),
        in_specs=[
            pl.BlockSpec((batch, emb), lambda i: (0, 0)),
            pl.BlockSpec((vblk, emb), lambda i: (i, 0)),
            pl.BlockSpec((1, vblk), lambda i: (0, i)),
        ],
        out_specs=pl.BlockSpec(memory_space=pl.ANY),
        out_shape=jax.ShapeDtypeStruct((batch, voc), jnp.float32),
        scratch_shapes=[
            pltpu.VMEM((nbuf, batch, vblk), jnp.float32),
            pltpu.SemaphoreType.DMA((nbuf,)),
        ],
        compiler_params=pltpu.CompilerParams(
            vmem_limit_bytes=100 * 1024 * 1024,
            dimension_semantics=("arbitrary",),
        ),
    )(pooled, lin_w, bias2d)

    done = nstep * vblk
    tail_blocks = pl.cdiv(voc - done, 128)
    first = done // 128
    return pl.pallas_call(
        _tail_body,
        grid=(tail_blocks,),
        in_specs=[
            pl.BlockSpec(memory_space=pl.ANY),
            pl.BlockSpec((batch, emb), lambda i: (0, 0)),
            pl.BlockSpec((128, emb), lambda i: (first + i, 0)),
            pl.BlockSpec((1, 128), lambda i: (0, first + i)),
        ],
        out_specs=pl.BlockSpec((batch, 128), lambda i: (0, first + i)),
        out_shape=jax.ShapeDtypeStruct((batch, voc), jnp.float32),
        input_output_aliases={0: 0},
        compiler_params=pltpu.CompilerParams(
            dimension_semantics=("arbitrary",),
        ),
    )(main, pooled, lin_w, bias2d)


def kernel(inpt, emb_table, lin_w, lin_b):
    batch, ctx = inpt.shape
    voc, emb = emb_table.shape
    idx = inpt.astype(jnp.int32).reshape(-1)
    pooled = _make_pool(batch, ctx, voc, emb)(idx, emb_table)
    return _project(pooled, lin_w, lin_b, vblk=2048, nbuf=4)


# speedup vs baseline: 2.2381x; 2.2381x over previous
import functools

import jax
import jax.numpy as jnp
from jax import lax
from jax.experimental import pallas as pl
from jax.experimental.pallas import tpu as pltpu
from jax.experimental.pallas import tpu_sc as plsc

_LANES = 16


def _make_pool(batch, ctx, voc, emb):
    info = plsc.get_sparse_core_info()
    nw = info.num_cores * info.num_subcores
    b_per_w = batch // nw
    idx_per_w = b_per_w * ctx
    nc = info.num_cores
    scale = jnp.float32(1.0 / ctx)

    def body(idx_hbm, table_hbm, out_hbm, idx_v, rows_v, pooled_v, sem):
        wid = lax.axis_index("s") * nc + lax.axis_index("c")
        ibase = wid * idx_per_w
        bbase = wid * b_per_w
        pltpu.sync_copy(idx_hbm.at[pl.ds(ibase, idx_per_w)], idx_v)
        pltpu.async_copy(table_hbm.at[idx_v], rows_v, sem).wait()

        def row_body(b, carry):
            def ctx_body(c, accs):
                a0, a1 = accs
                r = b * ctx + c
                return (a0 + rows_v[r, 0:_LANES],
                        a1 + rows_v[r, _LANES:2 * _LANES])

            a0, a1 = lax.fori_loop(
                0, ctx, ctx_body,
                (jnp.zeros((_LANES,), jnp.float32),
                 jnp.zeros((_LANES,), jnp.float32)))
            pooled_v[b, 0:_LANES] = a0 * scale
            pooled_v[b, _LANES:2 * _LANES] = a1 * scale
            return carry

        lax.fori_loop(0, b_per_w, row_body, 0)
        pltpu.sync_copy(pooled_v, out_hbm.at[pl.ds(bbase, b_per_w)])

    return pl.kernel(
        body,
        mesh=plsc.VectorSubcoreMesh(core_axis_name="c", subcore_axis_name="s"),
        compiler_params=pltpu.CompilerParams(use_tc_tiling_on_sc=False),
        out_type=jax.ShapeDtypeStruct((batch, emb), jnp.float32),
        scratch_types=[
            pltpu.VMEM((idx_per_w,), jnp.int32),
            pltpu.VMEM((idx_per_w, emb), jnp.float32),
            pltpu.VMEM((b_per_w, emb), jnp.float32),
            pltpu.SemaphoreType.DMA,
        ],
    )


_NBUF = 4


def _make_proj_body(vblk, nstep, tail):
    def body(p_ref, w_ref, b_ref, o_hbm, obuf, sems):
        i = pl.program_id(0)
        slot = lax.rem(i, _NBUF)

        @pl.when(i >= _NBUF)
        def _wait_prev():
            pltpu.make_async_copy(
                obuf.at[slot],
                o_hbm.at[:, pl.ds((i - _NBUF) * vblk, vblk)],
                sems.at[slot]).wait()

        obuf[slot] = lax.dot_general(
            p_ref[...], w_ref[...], (((1,), (1,)), ((), ())),
            preferred_element_type=jnp.float32) + b_ref[...]

        @pl.when(i < nstep - 1)
        def _fire():
            pltpu.make_async_copy(
                obuf.at[slot],
                o_hbm.at[:, pl.ds(i * vblk, vblk)],
                sems.at[slot]).start()

        @pl.when(i == nstep - 1)
        def _fire_tail_and_drain():
            pltpu.make_async_copy(
                obuf.at[slot, :, 0:tail],
                o_hbm.at[:, pl.ds((nstep - 1) * vblk, tail)],
                sems.at[slot]).start()
            for d in range(1, _NBUF):
                j = i - _NBUF + d
                sj = lax.rem(j, _NBUF)
                pltpu.make_async_copy(
                    obuf.at[sj],
                    o_hbm.at[:, pl.ds(j * vblk, vblk)],
                    sems.at[sj]).wait()
            pltpu.make_async_copy(
                obuf.at[slot, :, 0:tail],
                o_hbm.at[:, pl.ds((nstep - 1) * vblk, tail)],
                sems.at[slot]).wait()

    return body


def _make_main_body(vblk, nstep, nbuf, tail_rows):
    def body(pt_ref, wt_ref, b_ref, o_hbm, obuf, sems):
        i = pl.program_id(0)
        slot = lax.rem(i, nbuf)

        @pl.when(i >= nbuf)
        def _wait_prev():
            pltpu.make_async_copy(
                obuf.at[slot],
                o_hbm.at[pl.ds((i - nbuf) * vblk, vblk)],
                sems.at[slot]).wait()

        obuf[slot] = lax.dot_general(
            wt_ref[...], pt_ref[...], (((0,), (0,)), ((), ())),
            preferred_element_type=jnp.float32) + b_ref[...]

        @pl.when(i < nstep - 1)
        def _fire():
            pltpu.make_async_copy(
                obuf.at[slot],
                o_hbm.at[pl.ds(i * vblk, vblk)],
                sems.at[slot]).start()

        @pl.when(i == nstep - 1)
        def _fire_tail_and_drain():
            last = tail_rows
            pltpu.make_async_copy(
                obuf.at[slot, 0:last],
                o_hbm.at[pl.ds((nstep - 1) * vblk, last)],
                sems.at[slot]).start()
            for d in range(nbuf - 1, 0, -1):
                j = i - d
                sj = lax.rem(j, nbuf)
                pltpu.make_async_copy(
                    obuf.at[sj],
                    o_hbm.at[pl.ds(j * vblk, vblk)],
                    sems.at[sj]).wait()
            pltpu.make_async_copy(
                obuf.at[slot, 0:last],
                o_hbm.at[pl.ds((nstep - 1) * vblk, last)],
                sems.at[slot]).wait()

    return body


def _project(pooled, lin_w, lin_b, vblk, nbuf):
    batch, emb = pooled.shape
    voc = lin_w.shape[0]
    nstep = pl.cdiv(voc, vblk)
    tail_rows = voc - (nstep - 1) * vblk
    assert vblk % 128 == 0 and tail_rows % 8 == 0
    voc_pad = nstep * vblk
    wt = lin_w.T
    wt_pad = jnp.pad(wt, ((0, 0), (0, voc_pad - voc)))
    pt = pooled.T
    bias_col = lin_b.reshape(voc, 1)
    out_t = pl.pallas_call(
        _make_main_body(vblk, nstep, nbuf, tail_rows),
        grid=(nstep,),
        in_specs=[
            pl.BlockSpec((emb, batch), lambda i: (0, 0)),
            pl.BlockSpec((emb, vblk), lambda i: (0, i)),
            pl.BlockSpec((vblk, 1), lambda i: (i, 0)),
        ],
        out_specs=pl.BlockSpec(memory_space=pl.ANY),
        out_shape=jax.ShapeDtypeStruct((voc, batch), jnp.float32),
        scratch_shapes=[
            pltpu.VMEM((nbuf, vblk, batch), jnp.float32),
            pltpu.SemaphoreType.DMA((nbuf,)),
        ],
        compiler_params=pltpu.CompilerParams(
            vmem_limit_bytes=100 * 1024 * 1024,
            dimension_semantics=("arbitrary",),
        ),
    )(pt, wt_pad, bias_col)
    return out_t.T


def kernel(inpt, emb_table, lin_w, lin_b):
    batch, ctx = inpt.shape
    voc, emb = emb_table.shape
    idx = inpt.astype(jnp.int32).reshape(-1)
    pooled = _make_pool(batch, ctx, voc, emb)(idx, emb_table)
    return _project(pooled, lin_w, lin_b, vblk=2048, nbuf=6)


# speedup vs baseline: 2.8645x; 1.2799x over previous
import functools

import jax
import jax.numpy as jnp
from jax import lax
from jax.experimental import pallas as pl
from jax.experimental.pallas import tpu as pltpu
from jax.experimental.pallas import tpu_sc as plsc

_LANES = 16


def _make_pool(batch, ctx, voc, emb):
    info = plsc.get_sparse_core_info()
    nw = info.num_cores * info.num_subcores
    b_per_w = batch // nw
    idx_per_w = b_per_w * ctx
    nc = info.num_cores
    scale = jnp.float32(1.0 / ctx)

    def body(idx_hbm, table_hbm, out_hbm, idx_v, rows_v, pooled_v, sem):
        wid = lax.axis_index("s") * nc + lax.axis_index("c")
        ibase = wid * idx_per_w
        bbase = wid * b_per_w
        pltpu.sync_copy(idx_hbm.at[pl.ds(ibase, idx_per_w)], idx_v)
        pltpu.async_copy(table_hbm.at[idx_v], rows_v, sem).wait()

        def row_body(b, carry):
            def ctx_body(c, accs):
                a0, a1 = accs
                r = b * ctx + c
                return (a0 + rows_v[r, 0:_LANES],
                        a1 + rows_v[r, _LANES:2 * _LANES])

            a0, a1 = lax.fori_loop(
                0, ctx, ctx_body,
                (jnp.zeros((_LANES,), jnp.float32),
                 jnp.zeros((_LANES,), jnp.float32)))
            pooled_v[b, 0:_LANES] = a0 * scale
            pooled_v[b, _LANES:2 * _LANES] = a1 * scale
            return carry

        lax.fori_loop(0, b_per_w, row_body, 0)
        pltpu.sync_copy(pooled_v, out_hbm.at[pl.ds(bbase, b_per_w)])

    return pl.kernel(
        body,
        mesh=plsc.VectorSubcoreMesh(core_axis_name="c", subcore_axis_name="s"),
        compiler_params=pltpu.CompilerParams(use_tc_tiling_on_sc=False),
        out_type=jax.ShapeDtypeStruct((batch, emb), jnp.float32),
        scratch_types=[
            pltpu.VMEM((idx_per_w,), jnp.int32),
            pltpu.VMEM((idx_per_w, emb), jnp.float32),
            pltpu.VMEM((b_per_w, emb), jnp.float32),
            pltpu.SemaphoreType.DMA,
        ],
    )


_NBUF = 4


def _make_proj_body(vblk, nstep, tail):
    def body(p_ref, w_ref, b_ref, o_hbm, obuf, sems):
        i = pl.program_id(0)
        slot = lax.rem(i, _NBUF)

        @pl.when(i >= _NBUF)
        def _wait_prev():
            pltpu.make_async_copy(
                obuf.at[slot],
                o_hbm.at[:, pl.ds((i - _NBUF) * vblk, vblk)],
                sems.at[slot]).wait()

        obuf[slot] = lax.dot_general(
            p_ref[...], w_ref[...], (((1,), (1,)), ((), ())),
            preferred_element_type=jnp.float32) + b_ref[...]

        @pl.when(i < nstep - 1)
        def _fire():
            pltpu.make_async_copy(
                obuf.at[slot],
                o_hbm.at[:, pl.ds(i * vblk, vblk)],
                sems.at[slot]).start()

        @pl.when(i == nstep - 1)
        def _fire_tail_and_drain():
            pltpu.make_async_copy(
                obuf.at[slot, :, 0:tail],
                o_hbm.at[:, pl.ds((nstep - 1) * vblk, tail)],
                sems.at[slot]).start()
            for d in range(1, _NBUF):
                j = i - _NBUF + d
                sj = lax.rem(j, _NBUF)
                pltpu.make_async_copy(
                    obuf.at[sj],
                    o_hbm.at[:, pl.ds(j * vblk, vblk)],
                    sems.at[sj]).wait()
            pltpu.make_async_copy(
                obuf.at[slot, :, 0:tail],
                o_hbm.at[:, pl.ds((nstep - 1) * vblk, tail)],
                sems.at[slot]).wait()

    return body


def _make_main_body(vblk, nstep, nbuf, tail_rows):
    def body(pt_ref, wt_ref, o_hbm, obuf, sems):
        i = pl.program_id(0)
        slot = lax.rem(i, nbuf)

        @pl.when(i >= nbuf)
        def _wait_prev():
            pltpu.make_async_copy(
                obuf.at[slot],
                o_hbm.at[pl.ds((i - nbuf) * vblk, vblk)],
                sems.at[slot]).wait()

        obuf[slot] = lax.dot_general(
            wt_ref[...], pt_ref[...], (((0,), (0,)), ((), ())),
            preferred_element_type=jnp.float32)

        @pl.when(i < nstep - 1)
        def _fire():
            pltpu.make_async_copy(
                obuf.at[slot],
                o_hbm.at[pl.ds(i * vblk, vblk)],
                sems.at[slot]).start()

        @pl.when(i == nstep - 1)
        def _fire_tail_and_drain():
            last = tail_rows
            pltpu.make_async_copy(
                obuf.at[slot, 0:last],
                o_hbm.at[pl.ds((nstep - 1) * vblk, last)],
                sems.at[slot]).start()
            for d in range(nbuf - 1, 0, -1):
                j = i - d
                sj = lax.rem(j, nbuf)
                pltpu.make_async_copy(
                    obuf.at[sj],
                    o_hbm.at[pl.ds(j * vblk, vblk)],
                    sems.at[sj]).wait()
            pltpu.make_async_copy(
                obuf.at[slot, 0:last],
                o_hbm.at[pl.ds((nstep - 1) * vblk, last)],
                sems.at[slot]).wait()

    return body


def _project(pooled, lin_w, lin_b, vblk, nbuf):
    batch, emb = pooled.shape
    voc = lin_w.shape[0]
    nstep = pl.cdiv(voc, vblk)
    tail_rows = voc - (nstep - 1) * vblk
    assert vblk % 128 == 0 and tail_rows % 8 == 0
    voc_pad = nstep * vblk
    w_aug = jnp.pad(jnp.concatenate([lin_w.T, lin_b[None, :]], axis=0),
                    ((0, 0), (0, voc_pad - voc)))
    pt_aug = jnp.concatenate(
        [pooled.T, jnp.ones((1, batch), jnp.float32)], axis=0)
    out_t = pl.pallas_call(
        _make_main_body(vblk, nstep, nbuf, tail_rows),
        grid=(nstep,),
        in_specs=[
            pl.BlockSpec((emb + 1, batch), lambda i: (0, 0)),
            pl.BlockSpec((emb + 1, vblk), lambda i: (0, i)),
        ],
        out_specs=pl.BlockSpec(memory_space=pl.ANY),
        out_shape=jax.ShapeDtypeStruct((voc, batch), jnp.float32),
        scratch_shapes=[
            pltpu.VMEM((nbuf, vblk, batch), jnp.float32),
            pltpu.SemaphoreType.DMA((nbuf,)),
        ],
        compiler_params=pltpu.CompilerParams(
            vmem_limit_bytes=100 * 1024 * 1024,
            dimension_semantics=("arbitrary",),
        ),
    )(pt_aug, w_aug)
    return out_t.T


def kernel(inpt, emb_table, lin_w, lin_b):
    batch, ctx = inpt.shape
    voc, emb = emb_table.shape
    idx = inpt.astype(jnp.int32).reshape(-1)
    pooled = _make_pool(batch, ctx, voc, emb)(idx, emb_table)
    return _project(pooled, lin_w, lin_b, vblk=2048, nbuf=6)
